# Initial kernel scaffold; baseline (speedup 1.0000x reference)
import jax
import jax.numpy as jnp
from jax.experimental import pallas as pl


def kernel(x, edge_index, batch_ids, W_msg, W_self, enc_W, codebook, dec_W):
    raise NotImplementedError("write your pallas kernel here")


# speedup vs baseline: 2.6061x; 2.6061x over previous
import functools

import jax
import jax.numpy as jnp
from jax import lax
from jax.experimental import pallas as pl
from jax.experimental.pallas import tpu as pltpu
from jax.experimental.pallas import tpu_sc as plsc

N = 10000
E = 160000
D = 256
B = 16
L = 512
K = 1024
C = 64

DH = D // 2
BL = B * L
TRASH = BL
ACC_R = 8320
NP = 10240
XHALF = NP
CH = 128
NSUB = 16
EPS = 10112
E_PAD = EPS * NSUB
NCH = EPS // CH
PAD_CH = (BL // NSUB) // CH

ROWS_TC = 512
GRID_N = NP // ROWS_TC
GRID_TC = BL // ROWS_TC
LOSS_SCALE = 1.25 / float(BL * C)


def _mm_body(x_ref, wm_ref, ws_ref, m_ref, xw_ref):
    dot = functools.partial(
        jnp.dot, preferred_element_type=jnp.float32,
        precision=lax.Precision.DEFAULT)
    xb = x_ref[...]
    m_ref[...] = dot(xb, wm_ref[...])
    xw_ref[...] = dot(xb, ws_ref[...])


_mm_kernel = pl.pallas_call(
    _mm_body,
    grid=(2, GRID_N),
    in_specs=[
        pl.BlockSpec((ROWS_TC, D), lambda j, i: (i, 0)),
        pl.BlockSpec((D, DH), lambda j, i: (0, j)),
        pl.BlockSpec((D, DH), lambda j, i: (0, j)),
    ],
    out_specs=[
        pl.BlockSpec((ROWS_TC, DH), lambda j, i: (j * GRID_N + i, 0)),
        pl.BlockSpec((ROWS_TC, DH), lambda j, i: (j * GRID_N + i, 0)),
    ],
    out_shape=[
        jax.ShapeDtypeStruct((2 * NP, DH), jnp.float32),
        jax.ShapeDtypeStruct((2 * NP, DH), jnp.float32),
    ],
)


def _sc_body(m2, xw2, srcp, dstp, slotmap, gidx, zinit, a_out, xp_out,
             sbuf, dbuf, slotbuf, gbuf, xrows, acc, sem):
    c = lax.axis_index("c")
    s = lax.axis_index("s")
    half_base = c * XHALF

    pltpu.sync_copy(zinit, acc.at[pl.ds(s * (ACC_R // NSUB), ACC_R // NSUB)])
    plsc.subcore_barrier()

    ebase = s * EPS

    def edge_chunk(k, carry):
        off = pl.multiple_of(ebase + k * CH, 8)
        pltpu.sync_copy(srcp.at[pl.ds(off, CH)], sbuf)
        pltpu.sync_copy(dstp.at[pl.ds(off, CH)], dbuf)
        pltpu.async_copy(slotmap.at[dbuf], slotbuf, sem).wait()
        for j in range(CH // 16):
            sv = sbuf[pl.ds(j * 16, 16)]
            gbuf[pl.ds(j * 16, 16)] = sv + half_base
        pltpu.async_copy(m2.at[gbuf], xrows, sem).wait()
        pltpu.sync_copy(xrows, acc.at[slotbuf], add=True)
        return carry

    lax.fori_loop(0, NCH, edge_chunk, 0)
    plsc.subcore_barrier()

    rbase = s * (BL // NSUB)
    pltpu.sync_copy(acc.at[pl.ds(rbase, BL // NSUB)],
                    a_out.at[c, pl.ds(rbase, BL // NSUB)])

    def pad_chunk(t, carry):
        roff = pl.multiple_of(rbase + t * CH, 8)
        pltpu.sync_copy(gidx.at[pl.ds(roff, CH)], sbuf)
        for j in range(CH // 16):
            gbuf[pl.ds(j * 16, 16)] = sbuf[pl.ds(j * 16, 16)] + half_base
        pltpu.async_copy(xw2.at[gbuf], xrows, sem).wait()
        pltpu.sync_copy(xrows, xp_out.at[c, pl.ds(roff, CH)])
        return carry

    lax.fori_loop(0, PAD_CH, pad_chunk, 0)


@functools.cache
def _sc_kernel():
    return functools.partial(
        pl.kernel,
        mesh=plsc.VectorSubcoreMesh(core_axis_name="c", subcore_axis_name="s"),
        out_type=[
            jax.ShapeDtypeStruct((2, BL, DH), jnp.float32),
            jax.ShapeDtypeStruct((2, BL, DH), jnp.float32),
        ],
        scratch_types=[
            pltpu.VMEM((CH,), jnp.int32),
            pltpu.VMEM((CH,), jnp.int32),
            pltpu.VMEM((CH,), jnp.int32),
            pltpu.VMEM((CH,), jnp.int32),
            pltpu.VMEM((CH, DH), jnp.float32),
            pltpu.VMEM_SHARED((ACC_R, DH), jnp.float32),
            pltpu.SemaphoreType.DMA,
        ],
    )(_sc_body)


def _tc_body(a_ref, xp_ref, enc_ref, cb_ref, dec_ref,
             out_ref, idx_ref, loss_ref):
    i = pl.program_id(0)
    dot = functools.partial(
        jnp.dot, preferred_element_type=jnp.float32,
        precision=lax.Precision.DEFAULT)
    h = jnp.concatenate(
        [a_ref[0] + xp_ref[0], a_ref[1] + xp_ref[1]], axis=1)
    h = jnp.maximum(h, 0.0)
    z = dot(h, enc_ref[...])
    cb = cb_ref[...]
    zs = jnp.sum(z * z, axis=1, keepdims=True)
    cbs = jnp.sum(cb * cb, axis=1)
    sc = lax.dot_general(z, cb, (((1,), (1,)), ((), ())),
                         preferred_element_type=jnp.float32,
                         precision=lax.Precision.DEFAULT)
    d2 = zs - 2.0 * sc + cbs[None, :]
    m = jnp.min(d2, axis=1, keepdims=True)
    iot = lax.broadcasted_iota(jnp.int32, (ROWS_TC, K), 1)
    idx = jnp.min(jnp.where(d2 <= m, iot, K), axis=1)
    oh = (iot == idx[:, None]).astype(jnp.float32)
    q = jnp.dot(oh, cb, preferred_element_type=jnp.float32,
                precision=lax.Precision.HIGHEST)
    zq = z + (q - z)
    out_ref[...] = dot(zq, dec_ref[...])
    idx_ref[0, 0] = idx
    part = (jnp.sum((q - z) * (q - z)) * LOSS_SCALE).reshape(1, 1)

    @pl.when(i == 0)
    def _():
        loss_ref[...] = part

    @pl.when(i > 0)
    def _():
        loss_ref[...] += part


_tc_kernel = pl.pallas_call(
    _tc_body,
    grid=(GRID_TC,),
    in_specs=[
        pl.BlockSpec((2, ROWS_TC, DH), lambda i: (0, i, 0)),
        pl.BlockSpec((2, ROWS_TC, DH), lambda i: (0, i, 0)),
        pl.BlockSpec((D, C), lambda i: (0, 0)),
        pl.BlockSpec((K, C), lambda i: (0, 0)),
        pl.BlockSpec((C, D), lambda i: (0, 0)),
    ],
    out_specs=[
        pl.BlockSpec((ROWS_TC, D), lambda i: (i, 0)),
        pl.BlockSpec((1, 1, ROWS_TC), lambda i: (i, 0, 0)),
        pl.BlockSpec((1, 1), lambda i: (0, 0)),
    ],
    out_shape=[
        jax.ShapeDtypeStruct((BL, D), jnp.float32),
        jax.ShapeDtypeStruct((GRID_TC, 1, ROWS_TC), jnp.int32),
        jax.ShapeDtypeStruct((1, 1), jnp.float32),
    ],
)


@jax.jit
def kernel(x, edge_index, batch_ids, W_msg, W_self, enc_W, codebook, dec_W):
    bids = batch_ids.astype(jnp.int32)
    offsets = jnp.searchsorted(
        bids, jnp.arange(B, dtype=jnp.int32), side="left").astype(jnp.int32)
    pos = jnp.arange(N, dtype=jnp.int32) - offsets[bids]
    slot_map = jnp.where(pos < L, bids * L + pos, TRASH).astype(jnp.int32)

    counts = jnp.concatenate(
        [offsets[1:], jnp.array([N], jnp.int32)]) - offsets
    ll = jnp.arange(L, dtype=jnp.int32)
    gidx = jnp.where(ll[None, :] < counts[:, None],
                     offsets[:, None] + ll[None, :], N).reshape(BL)

    src_p = jnp.concatenate(
        [edge_index[0].astype(jnp.int32),
         jnp.full((E_PAD - E,), N, jnp.int32)])
    dst_p = jnp.concatenate(
        [edge_index[1].astype(jnp.int32),
         jnp.zeros((E_PAD - E,), jnp.int32)])

    xpad = jnp.pad(x, ((0, NP - N), (0, 0)))
    zinit = jnp.zeros((ACC_R // NSUB, DH), jnp.float32)

    m2, xw2 = _mm_kernel(xpad, W_msg, W_self)

    a_h, xp_h = _sc_kernel()(m2, xw2, src_p, dst_p, slot_map, gidx, zinit)

    out8, idx3, loss = _tc_kernel(a_h, xp_h, enc_W, codebook, dec_W)
    return (out8.reshape(B, L, D), idx3.reshape(B, L),
            loss.reshape(()))


# speedup vs baseline: 2.9811x; 1.1439x over previous
import functools

import jax
import jax.numpy as jnp
from jax import lax
from jax.experimental import pallas as pl
from jax.experimental.pallas import tpu as pltpu
from jax.experimental.pallas import tpu_sc as plsc

N = 10000
E = 160000
D = 256
B = 16
L = 512
K = 1024
C = 64

DH = D // 2
BL = B * L
TRASH = BL
ACC_R = 8320
NP = 10240
XHALF = NP
CH = 128
NSUB = 16
EPS = 10240
E_PAD = EPS * NSUB
NCH = EPS // CH
NMAC = NCH // 2
PAD_CH = (BL // NSUB) // CH

ROWS_TC = 512
GRID_N = NP // ROWS_TC
GRID_TC = BL // ROWS_TC
LOSS_SCALE = 1.25 / float(BL * C)


def _mm_body(x_ref, wm_ref, ws_ref, m_ref, xw_ref):
    dot = functools.partial(
        jnp.dot, preferred_element_type=jnp.float32,
        precision=lax.Precision.DEFAULT)
    xb = x_ref[...]
    m_ref[...] = dot(xb, wm_ref[...])
    xw_ref[...] = dot(xb, ws_ref[...])


_mm_kernel = pl.pallas_call(
    _mm_body,
    grid=(2, GRID_N),
    in_specs=[
        pl.BlockSpec((ROWS_TC, D), lambda j, i: (i, 0)),
        pl.BlockSpec((D, DH), lambda j, i: (0, j)),
        pl.BlockSpec((D, DH), lambda j, i: (0, j)),
    ],
    out_specs=[
        pl.BlockSpec((ROWS_TC, DH), lambda j, i: (j * GRID_N + i, 0)),
        pl.BlockSpec((ROWS_TC, DH), lambda j, i: (j * GRID_N + i, 0)),
    ],
    out_shape=[
        jax.ShapeDtypeStruct((2 * NP, DH), jnp.float32),
        jax.ShapeDtypeStruct((2 * NP, DH), jnp.float32),
    ],
)


def _sc_body(m2, xw2, srcp, dstp, slotmap, gidx, zinit, a_out, xp_out,
             sbuf0, dbuf0, slotbuf0, gbuf0, xrows0,
             sbuf1, dbuf1, slotbuf1, gbuf1, xrows1,
             acc, semA0, semA1, semB0, semB1, semC0, semC1):
    c = lax.axis_index("c")
    s = lax.axis_index("s")
    half_base = c * XHALF

    pltpu.sync_copy(zinit, acc.at[pl.ds(s * (ACC_R // NSUB), ACC_R // NSUB)])
    plsc.subcore_barrier()

    ebase = s * EPS
    bufs = ((sbuf0, dbuf0, slotbuf0, gbuf0, xrows0, semA0, semB0, semC0),
            (sbuf1, dbuf1, slotbuf1, gbuf1, xrows1, semA1, semB1, semC1))

    def issue_ids(t, u):
        sb, db = bufs[u][0], bufs[u][1]
        off = pl.multiple_of(ebase + (2 * t + u) * CH, 8)
        pltpu.async_copy(srcp.at[pl.ds(off, CH)], sb, bufs[u][5])
        pltpu.async_copy(dstp.at[pl.ds(off, CH)], db, bufs[u][5])

    issue_ids(0, 0)
    issue_ids(0, 1)

    def macro(t, carry):
        for u in (0, 1):
            sb, db, slb, gb, xr, sA, sB, sC = bufs[u]
            pltpu.make_async_copy(srcp.at[pl.ds(ebase, CH)], sb, sA).wait()
            pltpu.make_async_copy(dstp.at[pl.ds(ebase, CH)], db, sA).wait()
            for j in range(CH // 16):
                gb[pl.ds(j * 16, 16)] = sb[pl.ds(j * 16, 16)] + half_base
            @pl.when(t > 0)
            def _():
                pltpu.make_async_copy(xr, acc.at[slb], sC).wait()
            pltpu.async_copy(slotmap.at[db], slb, sB)
            pltpu.async_copy(m2.at[gb], xr, sB)

        for u in (0, 1):
            sb, db, slb, gb, xr, sA, sB, sC = bufs[u]
            pltpu.make_async_copy(slotmap.at[db], slb, sB).wait()
            pltpu.make_async_copy(m2.at[gb], xr, sB).wait()

            @pl.when(t + 1 < NMAC)
            def _():
                issue_ids(t + 1, u)
            pltpu.async_copy(xr, acc.at[slb], sC, add=True)
        return carry

    lax.fori_loop(0, NMAC, macro, 0)
    for u in (0, 1):
        _, _, slb, _, xr, _, _, sC = bufs[u]
        pltpu.make_async_copy(xr, acc.at[slb], sC).wait()
    plsc.subcore_barrier()

    rbase = s * (BL // NSUB)
    pltpu.async_copy(acc.at[pl.ds(rbase, BL // NSUB)],
                     a_out.at[c, pl.ds(rbase, BL // NSUB)], semA0)

    def pad_gather(t, u):
        sb, _, _, gb, xr, _, sB, _ = bufs[u]
        roff = pl.multiple_of(rbase + t * CH, 8)
        pltpu.sync_copy(gidx.at[pl.ds(roff, CH)], sb)
        for j in range(CH // 16):
            gb[pl.ds(j * 16, 16)] = sb[pl.ds(j * 16, 16)] + half_base
        return pltpu.async_copy(xw2.at[gb], xr, sB)

    g = pad_gather(0, 0)
    writes = []
    for t in range(PAD_CH):
        u = t % 2
        g.wait()
        if t + 1 < PAD_CH:
            g = pad_gather(t + 1, (t + 1) % 2)
        roff = pl.multiple_of(rbase + t * CH, 8)
        writes.append(pltpu.async_copy(
            bufs[u][4], xp_out.at[c, pl.ds(roff, CH)], bufs[u][7]))
        if len(writes) >= 2:
            writes.pop(0).wait()
    for w in writes:
        w.wait()
    pltpu.make_async_copy(acc.at[pl.ds(rbase, BL // NSUB)],
                          a_out.at[c, pl.ds(rbase, BL // NSUB)], semA0).wait()


@functools.cache
def _sc_kernel():
    ibuf = pltpu.VMEM((CH,), jnp.int32)
    rbuf = pltpu.VMEM((CH, DH), jnp.float32)
    return functools.partial(
        pl.kernel,
        mesh=plsc.VectorSubcoreMesh(core_axis_name="c", subcore_axis_name="s"),
        out_type=[
            jax.ShapeDtypeStruct((2, BL, DH), jnp.float32),
            jax.ShapeDtypeStruct((2, BL, DH), jnp.float32),
        ],
        scratch_types=[
            ibuf, ibuf, ibuf, ibuf, rbuf,
            ibuf, ibuf, ibuf, ibuf, rbuf,
            pltpu.VMEM_SHARED((ACC_R, DH), jnp.float32),
            pltpu.SemaphoreType.DMA, pltpu.SemaphoreType.DMA,
            pltpu.SemaphoreType.DMA, pltpu.SemaphoreType.DMA,
            pltpu.SemaphoreType.DMA, pltpu.SemaphoreType.DMA,
        ],
    )(_sc_body)


def _tc_body(a_ref, xp_ref, enc_ref, cb_ref, dec_ref,
             out_ref, idx_ref, loss_ref):
    i = pl.program_id(0)
    dot = functools.partial(
        jnp.dot, preferred_element_type=jnp.float32,
        precision=lax.Precision.DEFAULT)
    h = jnp.concatenate(
        [a_ref[0] + xp_ref[0], a_ref[1] + xp_ref[1]], axis=1)
    h = jnp.maximum(h, 0.0)
    z = dot(h, enc_ref[...])
    cb = cb_ref[...]
    zs = jnp.sum(z * z, axis=1, keepdims=True)
    cbs = jnp.sum(cb * cb, axis=1)
    sc = lax.dot_general(z, cb, (((1,), (1,)), ((), ())),
                         preferred_element_type=jnp.float32,
                         precision=lax.Precision.DEFAULT)
    d2 = zs - 2.0 * sc + cbs[None, :]
    m = jnp.min(d2, axis=1, keepdims=True)
    iot = lax.broadcasted_iota(jnp.int32, (ROWS_TC, K), 1)
    idx = jnp.min(jnp.where(d2 <= m, iot, K), axis=1)
    oh = (iot == idx[:, None]).astype(jnp.float32)
    q = jnp.dot(oh, cb, preferred_element_type=jnp.float32,
                precision=lax.Precision.HIGHEST)
    zq = z + (q - z)
    out_ref[...] = dot(zq, dec_ref[...])
    idx_ref[0, 0] = idx
    part = (jnp.sum((q - z) * (q - z)) * LOSS_SCALE).reshape(1, 1)

    @pl.when(i == 0)
    def _():
        loss_ref[...] = part

    @pl.when(i > 0)
    def _():
        loss_ref[...] += part


_tc_kernel = pl.pallas_call(
    _tc_body,
    grid=(GRID_TC,),
    in_specs=[
        pl.BlockSpec((2, ROWS_TC, DH), lambda i: (0, i, 0)),
        pl.BlockSpec((2, ROWS_TC, DH), lambda i: (0, i, 0)),
        pl.BlockSpec((D, C), lambda i: (0, 0)),
        pl.BlockSpec((K, C), lambda i: (0, 0)),
        pl.BlockSpec((C, D), lambda i: (0, 0)),
    ],
    out_specs=[
        pl.BlockSpec((ROWS_TC, D), lambda i: (i, 0)),
        pl.BlockSpec((1, 1, ROWS_TC), lambda i: (i, 0, 0)),
        pl.BlockSpec((1, 1), lambda i: (0, 0)),
    ],
    out_shape=[
        jax.ShapeDtypeStruct((BL, D), jnp.float32),
        jax.ShapeDtypeStruct((GRID_TC, 1, ROWS_TC), jnp.int32),
        jax.ShapeDtypeStruct((1, 1), jnp.float32),
    ],
)


@jax.jit
def kernel(x, edge_index, batch_ids, W_msg, W_self, enc_W, codebook, dec_W):
    bids = batch_ids.astype(jnp.int32)
    offsets = jnp.searchsorted(
        bids, jnp.arange(B, dtype=jnp.int32), side="left").astype(jnp.int32)
    pos = jnp.arange(N, dtype=jnp.int32) - offsets[bids]
    slot_map = jnp.where(pos < L, bids * L + pos, TRASH).astype(jnp.int32)

    counts = jnp.concatenate(
        [offsets[1:], jnp.array([N], jnp.int32)]) - offsets
    ll = jnp.arange(L, dtype=jnp.int32)
    gidx = jnp.where(ll[None, :] < counts[:, None],
                     offsets[:, None] + ll[None, :], N).reshape(BL)

    src_p = jnp.concatenate(
        [edge_index[0].astype(jnp.int32),
         jnp.full((E_PAD - E,), N, jnp.int32)])
    dst_p = jnp.concatenate(
        [edge_index[1].astype(jnp.int32),
         jnp.zeros((E_PAD - E,), jnp.int32)])

    xpad = jnp.pad(x, ((0, NP - N), (0, 0)))
    zinit = jnp.zeros((ACC_R // NSUB, DH), jnp.float32)

    m2, xw2 = _mm_kernel(xpad, W_msg, W_self)

    a_h, xp_h = _sc_kernel()(m2, xw2, src_p, dst_p, slot_map, gidx, zinit)

    out8, idx3, loss = _tc_kernel(a_h, xp_h, enc_W, codebook, dec_W)
    return (out8.reshape(B, L, D), idx3.reshape(B, L),
            loss.reshape(()))
